# Initial kernel scaffold; baseline (speedup 1.0000x reference)
import functools

import jax
import jax.numpy as jnp
from jax import lax
from jax.experimental import pallas as pl
from jax.experimental.pallas import tpu as pltpu
from jax.experimental.pallas import tpu_sc as plsc

N = 10000
E = 320000
D = 128

NC = 2
NS = 16
NW = NC * NS

E_W = E // NW
CH = 128
N_FULL = E_W // CH
TAIL = E_W - N_FULL * CH

N_PAD = 10240
ZROWS = 125

_mesh = plsc.VectorSubcoreMesh(
    core_axis_name="c", subcore_axis_name="s", num_cores=NC, num_subcores=NS
)


def _fill_1d(ref, n, value):
    v = jnp.full((16,), value, jnp.float32)

    def body(i, _):
        ref[pl.ds(i * 16, 16)] = v
        return 0

    lax.fori_loop(0, n // 16, body, 0)


@functools.partial(
    pl.kernel,
    out_type=jax.ShapeDtypeStruct((NC, N), jnp.float32),
    mesh=_mesh,
    scratch_types=[
        pltpu.VMEM((CH,), jnp.int32),
        pltpu.VMEM((TAIL,), jnp.int32),
        pltpu.VMEM((CH,), jnp.float32),
        pltpu.VMEM((640,), jnp.float32),
        pltpu.VMEM_SHARED((N_PAD,), jnp.float32),
    ],
)
def _deg(rows_hbm, out_hbm, idx_v, idx_t, ones_v, zeros_v, deg_sh):
    cid = lax.axis_index("c")
    sid = lax.axis_index("s")
    wid = sid * NC + cid

    _fill_1d(ones_v, CH, 1.0)
    _fill_1d(zeros_v, 640, 0.0)
    pltpu.sync_copy(zeros_v, deg_sh.at[pl.ds(sid * 640, 640)])
    plsc.subcore_barrier()

    wb = wid * E_W

    def chunk(j, _):
        base = pl.multiple_of(wb + j * CH, 8)
        pltpu.sync_copy(rows_hbm.at[pl.ds(base, CH)], idx_v)
        pltpu.sync_copy(ones_v, deg_sh.at[idx_v], add=True)
        return 0

    lax.fori_loop(0, N_FULL, chunk, 0)
    tbase = pl.multiple_of(wb + N_FULL * CH, 8)
    pltpu.sync_copy(rows_hbm.at[pl.ds(tbase, TAIL)], idx_t)
    pltpu.sync_copy(ones_v.at[pl.ds(0, TAIL)], deg_sh.at[idx_t], add=True)

    plsc.subcore_barrier()

    @pl.when(sid == 0)
    def _():
        pltpu.sync_copy(deg_sh.at[pl.ds(0, N)], out_hbm.at[cid])


@functools.partial(
    pl.kernel,
    out_type=jax.ShapeDtypeStruct((NC, N, D), jnp.float32),
    mesh=_mesh,
    scratch_types=[
        pltpu.VMEM((CH,), jnp.int32),
        pltpu.VMEM((CH,), jnp.int32),
        pltpu.VMEM((TAIL,), jnp.int32),
        pltpu.VMEM((TAIL,), jnp.int32),
        pltpu.VMEM((CH, D), jnp.float32),
        pltpu.VMEM((TAIL, D), jnp.float32),
        pltpu.VMEM((ZROWS, D), jnp.float32),
        pltpu.VMEM_SHARED((N, D), jnp.float32),
        pltpu.SemaphoreType.DMA,
    ],
)
def _spmm(rows_hbm, cols_hbm, g_hbm, acc_hbm,
          colv, rowv, colv_t, rowv_t, gbuf, gbuf_t, zbuf, acc_sh, sem):
    cid = lax.axis_index("c")
    sid = lax.axis_index("s")
    wid = sid * NC + cid

    def zrow(i, _):
        def zcol(j, _):
            zbuf[i, pl.ds(j * 16, 16)] = jnp.zeros((16,), jnp.float32)
            return 0
        lax.fori_loop(0, D // 16, zcol, 0)
        return 0

    lax.fori_loop(0, ZROWS, zrow, 0)
    r0 = sid * (N // NS)
    for k in range(5):
        pltpu.sync_copy(zbuf, acc_sh.at[pl.ds(r0 + k * ZROWS, ZROWS)])
    plsc.subcore_barrier()

    wb = wid * E_W

    def chunk(j, _):
        base = pl.multiple_of(wb + j * CH, 8)
        pltpu.sync_copy(cols_hbm.at[pl.ds(base, CH)], colv)
        pltpu.sync_copy(rows_hbm.at[pl.ds(base, CH)], rowv)
        pltpu.async_copy(g_hbm.at[colv], gbuf, sem).wait()
        pltpu.sync_copy(gbuf, acc_sh.at[rowv], add=True)
        return 0

    lax.fori_loop(0, N_FULL, chunk, 0)
    tbase = pl.multiple_of(wb + N_FULL * CH, 8)
    pltpu.sync_copy(cols_hbm.at[pl.ds(tbase, TAIL)], colv_t)
    pltpu.sync_copy(rows_hbm.at[pl.ds(tbase, TAIL)], rowv_t)
    pltpu.async_copy(g_hbm.at[colv_t], gbuf_t, sem).wait()
    pltpu.sync_copy(gbuf_t, acc_sh.at[rowv_t], add=True)

    plsc.subcore_barrier()
    pltpu.sync_copy(acc_sh.at[pl.ds(r0, N // NS)],
                    acc_hbm.at[cid, pl.ds(r0, N // NS)])


_BLK = 1000


def _linear_body(x_ref, wt_ref, b_ref, deg_ref, g_ref):
    h = jnp.dot(x_ref[...], wt_ref[...], preferred_element_type=jnp.float32)
    h = h + b_ref[...]
    deg = jnp.sum(deg_ref[...], axis=1, keepdims=True) + 1.0
    g_ref[...] = h * lax.rsqrt(deg)


def _combine_body(acc_ref, g_ref, deg_ref, o_ref):
    deg = jnp.sum(deg_ref[...], axis=1, keepdims=True) + 1.0
    o_ref[...] = (acc_ref[0] + acc_ref[1] + g_ref[...]) * lax.rsqrt(deg)


def kernel(x, edge_index, W, b):
    ei = edge_index.astype(jnp.int32)
    rows, cols = ei[0], ei[1]

    degs = _deg(rows)
    degs_t = degs.T

    g = pl.pallas_call(
        _linear_body,
        grid=(N // _BLK,),
        in_specs=[
            pl.BlockSpec((_BLK, D), lambda i: (i, 0)),
            pl.BlockSpec((D, D), lambda i: (0, 0)),
            pl.BlockSpec((1, D), lambda i: (0, 0)),
            pl.BlockSpec((_BLK, 2), lambda i: (i, 0)),
        ],
        out_specs=pl.BlockSpec((_BLK, D), lambda i: (i, 0)),
        out_shape=jax.ShapeDtypeStruct((N, D), jnp.float32),
    )(x, W.T, b.reshape(1, D), degs_t)

    accs = _spmm(rows, cols, g)

    out = pl.pallas_call(
        _combine_body,
        grid=(N // _BLK,),
        in_specs=[
            pl.BlockSpec((NC, _BLK, D), lambda i: (0, i, 0)),
            pl.BlockSpec((_BLK, D), lambda i: (i, 0)),
            pl.BlockSpec((_BLK, 2), lambda i: (i, 0)),
        ],
        out_specs=pl.BlockSpec((_BLK, D), lambda i: (i, 0)),
        out_shape=jax.ShapeDtypeStruct((N, D), jnp.float32),
    )(accs, g, degs_t)
    return out


# speedup vs baseline: 22.5684x; 22.5684x over previous
import functools

import jax
import jax.numpy as jnp
from jax import lax
from jax.experimental import pallas as pl
from jax.experimental.pallas import tpu as pltpu
from jax.experimental.pallas import tpu_sc as plsc

N = 10000
E = 320000
D = 128

NC = 2
NS = 16
NW = NC * NS

E_W = E // NW
CH = 128
N_FULL = E_W // CH
TAIL = E_W - N_FULL * CH

N_PAD = 10240
ZROWS = 125

_mesh = plsc.VectorSubcoreMesh(
    core_axis_name="c", subcore_axis_name="s", num_cores=NC, num_subcores=NS
)


def _fill_1d(ref, n, value):
    v = jnp.full((16,), value, jnp.float32)

    def body(i, _):
        ref[pl.ds(i * 16, 16)] = v
        return 0

    lax.fori_loop(0, n // 16, body, 0)


@functools.partial(
    pl.kernel,
    out_type=jax.ShapeDtypeStruct((NC, N_PAD), jnp.float32),
    mesh=_mesh,
    scratch_types=[
        pltpu.VMEM((CH,), jnp.int32),
        pltpu.VMEM((TAIL,), jnp.int32),
        pltpu.VMEM((CH,), jnp.float32),
        pltpu.VMEM((640,), jnp.float32),
        pltpu.VMEM_SHARED((N_PAD,), jnp.float32),
    ],
)
def _deg(rows_hbm, out_hbm, idx_v, idx_t, ones_v, zeros_v, deg_sh):
    cid = lax.axis_index("c")
    sid = lax.axis_index("s")
    wid = sid * NC + cid

    _fill_1d(ones_v, CH, 1.0)
    _fill_1d(zeros_v, 640, 0.0)
    pltpu.sync_copy(zeros_v, deg_sh.at[pl.ds(sid * 640, 640)])
    plsc.subcore_barrier()

    wb = wid * E_W

    def chunk(j, _):
        base = pl.multiple_of(wb + j * CH, 8)
        pltpu.sync_copy(rows_hbm.at[pl.ds(base, CH)], idx_v)
        pltpu.sync_copy(ones_v, deg_sh.at[idx_v], add=True)
        return 0

    lax.fori_loop(0, N_FULL, chunk, 0)
    tbase = pl.multiple_of(wb + N_FULL * CH, 8)
    pltpu.sync_copy(rows_hbm.at[pl.ds(tbase, TAIL)], idx_t)
    pltpu.sync_copy(ones_v.at[pl.ds(0, TAIL)], deg_sh.at[idx_t], add=True)

    plsc.subcore_barrier()

    @pl.when(sid == 0)
    def _():
        pltpu.sync_copy(deg_sh, out_hbm.at[cid])


@functools.partial(
    pl.kernel,
    out_type=jax.ShapeDtypeStruct((NC, N, D), jnp.float32),
    mesh=_mesh,
    scratch_types=[
        pltpu.VMEM((CH,), jnp.int32),
        pltpu.VMEM((CH,), jnp.int32),
        pltpu.VMEM((TAIL,), jnp.int32),
        pltpu.VMEM((TAIL,), jnp.int32),
        pltpu.VMEM((CH, D), jnp.float32),
        pltpu.VMEM((TAIL, D), jnp.float32),
        pltpu.VMEM((ZROWS, D), jnp.float32),
        pltpu.VMEM_SHARED((N, D), jnp.float32),
        pltpu.SemaphoreType.DMA,
    ],
)
def _spmm(rows_hbm, cols_hbm, g_hbm, acc_hbm,
          colv, rowv, colv_t, rowv_t, gbuf, gbuf_t, zbuf, acc_sh, sem):
    cid = lax.axis_index("c")
    sid = lax.axis_index("s")
    wid = sid * NC + cid

    def zrow(i, _):
        def zcol(j, _):
            zbuf[i, pl.ds(j * 16, 16)] = jnp.zeros((16,), jnp.float32)
            return 0
        lax.fori_loop(0, D // 16, zcol, 0)
        return 0

    lax.fori_loop(0, ZROWS, zrow, 0)
    r0 = sid * (N // NS)
    for k in range(5):
        pltpu.sync_copy(zbuf, acc_sh.at[pl.ds(r0 + k * ZROWS, ZROWS)])
    plsc.subcore_barrier()

    wb = wid * E_W

    def chunk(j, _):
        base = pl.multiple_of(wb + j * CH, 8)
        pltpu.sync_copy(cols_hbm.at[pl.ds(base, CH)], colv)
        pltpu.sync_copy(rows_hbm.at[pl.ds(base, CH)], rowv)
        pltpu.async_copy(g_hbm.at[colv], gbuf, sem).wait()
        pltpu.sync_copy(gbuf, acc_sh.at[rowv], add=True)
        return 0

    lax.fori_loop(0, N_FULL, chunk, 0)
    tbase = pl.multiple_of(wb + N_FULL * CH, 8)
    pltpu.sync_copy(cols_hbm.at[pl.ds(tbase, TAIL)], colv_t)
    pltpu.sync_copy(rows_hbm.at[pl.ds(tbase, TAIL)], rowv_t)
    pltpu.async_copy(g_hbm.at[colv_t], gbuf_t, sem).wait()
    pltpu.sync_copy(gbuf_t, acc_sh.at[rowv_t], add=True)

    plsc.subcore_barrier()

    @pl.when(sid < NS - 1)
    def _():
        s0 = sid * 624
        pltpu.sync_copy(acc_sh.at[pl.ds(s0, 624)],
                        acc_hbm.at[cid, pl.ds(s0, 624)])

    @pl.when(sid == NS - 1)
    def _():
        pltpu.sync_copy(acc_sh.at[pl.ds(624 * (NS - 1), 640)],
                        acc_hbm.at[cid, pl.ds(624 * (NS - 1), 640)])


_BLK = 1000


def _linear_body(x_ref, wt_ref, b_ref, deg_ref, g_ref):
    h = jnp.dot(x_ref[...], wt_ref[...], preferred_element_type=jnp.float32)
    h = h + b_ref[...]
    deg = jnp.sum(deg_ref[...], axis=1, keepdims=True) + 1.0
    g_ref[...] = h * lax.rsqrt(deg)


def _combine_body(acc_ref, g_ref, deg_ref, o_ref):
    deg = jnp.sum(deg_ref[...], axis=1, keepdims=True) + 1.0
    o_ref[...] = (acc_ref[0] + acc_ref[1] + g_ref[...]) * lax.rsqrt(deg)


def kernel(x, edge_index, W, b):
    ei = edge_index.astype(jnp.int32)
    rows, cols = ei[0], ei[1]

    degs = _deg(rows)
    degs_t = degs[:, :N].T

    g = pl.pallas_call(
        _linear_body,
        grid=(N // _BLK,),
        in_specs=[
            pl.BlockSpec((_BLK, D), lambda i: (i, 0)),
            pl.BlockSpec((D, D), lambda i: (0, 0)),
            pl.BlockSpec((1, D), lambda i: (0, 0)),
            pl.BlockSpec((_BLK, 2), lambda i: (i, 0)),
        ],
        out_specs=pl.BlockSpec((_BLK, D), lambda i: (i, 0)),
        out_shape=jax.ShapeDtypeStruct((N, D), jnp.float32),
    )(x, W.T, b.reshape(1, D), degs_t)

    accs = _spmm(rows, cols, g)

    out = pl.pallas_call(
        _combine_body,
        grid=(N // _BLK,),
        in_specs=[
            pl.BlockSpec((NC, _BLK, D), lambda i: (0, i, 0)),
            pl.BlockSpec((_BLK, D), lambda i: (i, 0)),
            pl.BlockSpec((_BLK, 2), lambda i: (i, 0)),
        ],
        out_specs=pl.BlockSpec((_BLK, D), lambda i: (i, 0)),
        out_shape=jax.ShapeDtypeStruct((N, D), jnp.float32),
    )(accs, g, degs_t)
    return out


# speedup vs baseline: 46.3951x; 2.0558x over previous
import functools

import jax
import jax.numpy as jnp
from jax import lax
from jax.experimental import pallas as pl
from jax.experimental.pallas import tpu as pltpu
from jax.experimental.pallas import tpu_sc as plsc

N = 10000
E = 320000
D = 128

NC = 2
NS = 16
NW = NC * NS

E_W = E // NW
CH = 125
NCHUNK = E_W // CH

N_PAD = 10240
NBUF = 2
GRP = 8
DEG_WIN = 16

_mesh = plsc.VectorSubcoreMesh(
    core_axis_name="c", subcore_axis_name="s", num_cores=NC, num_subcores=NS
)


def _fill_1d(ref, n, value):
    v = jnp.full((16,), value, jnp.float32)

    def body(i, _):
        ref[pl.ds(i * 16, 16)] = v
        return 0

    lax.fori_loop(0, n // 16, body, 0)


@functools.partial(
    pl.kernel,
    out_type=jax.ShapeDtypeStruct((NC, N_PAD), jnp.float32),
    mesh=_mesh,
    scratch_types=[
        pltpu.VMEM((NCHUNK, CH), jnp.int32),
        pltpu.VMEM((128,), jnp.float32),
        pltpu.VMEM((640,), jnp.float32),
        pltpu.VMEM_SHARED((N_PAD,), jnp.float32),
        pltpu.SemaphoreType.DMA,
    ],
)
def _deg(rows_hbm, out_hbm, idx2, ones_v, zeros_v, deg_sh, sem):
    cid = lax.axis_index("c")
    sid = lax.axis_index("s")
    wid = sid * NC + cid

    _fill_1d(ones_v, 128, 1.0)
    _fill_1d(zeros_v, 640, 0.0)
    pltpu.sync_copy(zeros_v, deg_sh.at[pl.ds(sid * 640, 640)])
    pltpu.sync_copy(rows_hbm.at[wid], idx2)
    plsc.subcore_barrier()

    ones_src = ones_v.at[pl.ds(0, CH)]

    def prime(j, _):
        pltpu.async_copy(ones_src, deg_sh.at[idx2.at[j]], sem, add=True)
        return 0

    lax.fori_loop(0, DEG_WIN, prime, 0)

    def step(j, _):
        pltpu.make_async_copy(ones_src, deg_sh.at[idx2.at[j]], sem).wait()

        @pl.when(j < NCHUNK - DEG_WIN)
        def _():
            pltpu.async_copy(ones_src, deg_sh.at[idx2.at[j + DEG_WIN]], sem,
                             add=True)

        return 0

    lax.fori_loop(0, NCHUNK, step, 0)

    plsc.subcore_barrier()

    @pl.when(sid == 0)
    def _():
        pltpu.sync_copy(deg_sh, out_hbm.at[cid])


@functools.partial(
    pl.kernel,
    out_type=jax.ShapeDtypeStruct((NC, N, D), jnp.float32),
    mesh=_mesh,
    scratch_types=[
        pltpu.VMEM((NCHUNK, CH), jnp.int32),
        [pltpu.VMEM((GRP, CH), jnp.int32)] * 2,
        [pltpu.VMEM((CH, D), jnp.float32)] * NBUF,
        [pltpu.SemaphoreType.DMA] * NBUF,
        [pltpu.SemaphoreType.DMA] * 2,
        pltpu.SemaphoreType.DMA,
        pltpu.VMEM_SHARED((N, D), jnp.float32),
    ],
)
def _spmm(rows_hbm, cols_hbm, g_hbm, acc_hbm,
          colv2, rowbs, gbufs, gsems, rsems, isem, acc_sh):
    cid = lax.axis_index("c")
    sid = lax.axis_index("s")
    wid = sid * NC + cid
    ngrp = NCHUNK // GRP

    def rows_src(g):
        return rows_hbm.at[wid, pl.ds(pl.multiple_of(g * GRP, GRP), GRP)]

    c_idx = pltpu.async_copy(cols_hbm.at[wid], colv2, isem)
    for p in range(2):
        pltpu.async_copy(rows_src(p), rowbs[p], rsems[p])

    zb = gbufs[0]

    def zrow(i, _):
        def zcol(j, _):
            zb[i, pl.ds(j * 16, 16)] = jnp.zeros((16,), jnp.float32)
            return 0
        lax.fori_loop(0, D // 16, zcol, 0)
        return 0

    lax.fori_loop(0, CH, zrow, 0)
    r0 = sid * (N // NS)
    for k in range(5):
        pltpu.sync_copy(zb, acc_sh.at[pl.ds(r0 + k * CH, CH)])
    c_idx.wait()
    plsc.subcore_barrier()

    for b in range(NBUF):
        pltpu.async_copy(g_hbm.at[colv2.at[b]], gbufs[b], gsems[b])

    def group(g, p):
        pltpu.make_async_copy(rows_src(g), rowbs[p], rsems[p]).wait()
        for i in range(GRP):
            b = i % NBUF
            j = g * GRP + i
            pltpu.make_async_copy(g_hbm.at[colv2.at[j]], gbufs[b],
                                  gsems[b]).wait()
            pltpu.sync_copy(gbufs[b], acc_sh.at[rowbs[p].at[i]], add=True)

            @pl.when(j < NCHUNK - NBUF)
            def _(j=j, b=b):
                pltpu.async_copy(g_hbm.at[colv2.at[j + NBUF]], gbufs[b],
                                 gsems[b])

        @pl.when(g < ngrp - 2)
        def _():
            pltpu.async_copy(rows_src(g + 2), rowbs[p], rsems[p])

    def step(kk, _):
        group(2 * kk, 0)
        group(2 * kk + 1, 1)
        return 0

    lax.fori_loop(0, ngrp // 2, step, 0)

    plsc.subcore_barrier()

    @pl.when(sid < NS - 1)
    def _():
        s0 = sid * 624
        pltpu.sync_copy(acc_sh.at[pl.ds(s0, 624)],
                        acc_hbm.at[cid, pl.ds(s0, 624)])

    @pl.when(sid == NS - 1)
    def _():
        pltpu.sync_copy(acc_sh.at[pl.ds(624 * (NS - 1), 640)],
                        acc_hbm.at[cid, pl.ds(624 * (NS - 1), 640)])


_BLK = 1000


def _linear_body(x_ref, wt_ref, b_ref, deg_ref, g_ref):
    h = jnp.dot(x_ref[...], wt_ref[...], preferred_element_type=jnp.float32)
    h = h + b_ref[...]
    deg = jnp.sum(deg_ref[...], axis=1, keepdims=True) + 1.0
    g_ref[...] = h * lax.rsqrt(deg)


def _combine_body(acc_ref, g_ref, deg_ref, o_ref):
    deg = jnp.sum(deg_ref[...], axis=1, keepdims=True) + 1.0
    o_ref[...] = (acc_ref[0] + acc_ref[1] + g_ref[...]) * lax.rsqrt(deg)


def kernel(x, edge_index, W, b):
    ei = edge_index.astype(jnp.int32)
    rows3 = ei[0].reshape(NW, NCHUNK, CH)
    cols3 = ei[1].reshape(NW, NCHUNK, CH)

    degs = _deg(rows3)
    degs_t = degs[:, :N].T

    g = pl.pallas_call(
        _linear_body,
        grid=(N // _BLK,),
        in_specs=[
            pl.BlockSpec((_BLK, D), lambda i: (i, 0)),
            pl.BlockSpec((D, D), lambda i: (0, 0)),
            pl.BlockSpec((1, D), lambda i: (0, 0)),
            pl.BlockSpec((_BLK, 2), lambda i: (i, 0)),
        ],
        out_specs=pl.BlockSpec((_BLK, D), lambda i: (i, 0)),
        out_shape=jax.ShapeDtypeStruct((N, D), jnp.float32),
    )(x, W.T, b.reshape(1, D), degs_t)

    accs = _spmm(rows3, cols3, g)

    out = pl.pallas_call(
        _combine_body,
        grid=(N // _BLK,),
        in_specs=[
            pl.BlockSpec((NC, _BLK, D), lambda i: (0, i, 0)),
            pl.BlockSpec((_BLK, D), lambda i: (i, 0)),
            pl.BlockSpec((_BLK, 2), lambda i: (i, 0)),
        ],
        out_specs=pl.BlockSpec((_BLK, D), lambda i: (i, 0)),
        out_shape=jax.ShapeDtypeStruct((N, D), jnp.float32),
    )(accs, g, degs_t)
    return out
